# baseline (speedup 1.0000x reference)
import functools

import jax
import jax.numpy as jnp
from jax.experimental import pallas as pl
from jax.experimental.pallas import tpu as pltpu


def _router_kernel(x_ref, w_ref, b_ref, out_ref, cnt_ref):
    d = x_ref.shape[1]
    ks = d // 4
    acc = jnp.dot(x_ref[:, 0:ks], w_ref[0:ks, :],
                  preferred_element_type=jnp.float32)
    for j in range(1, 4):
        acc = acc + jnp.dot(x_ref[:, j * ks:(j + 1) * ks],
                            w_ref[j * ks:(j + 1) * ks, :],
                            preferred_element_type=jnp.float32)
    logits = jnp.maximum(acc + b_ref[...], 0.0)
    out_ref[...] = logits
    nz = jnp.sum((logits > 0.0).astype(jnp.float32))
    cnt_ref[...] = jnp.full(cnt_ref.shape, nz, dtype=jnp.float32)


@functools.partial(jax.jit, static_argnames=("block_t",))
def _run(x, W, b, block_t):
    n_tokens, d_model = x.shape
    n_experts = W.shape[1]
    n_tiles = n_tokens // block_t
    b2 = b.reshape(1, n_experts)

    logits, counts = pl.pallas_call(
        _router_kernel,
        grid=(n_tiles,),
        in_specs=[
            pl.BlockSpec((block_t, d_model), lambda i: (i, 0)),
            pl.BlockSpec((d_model, n_experts), lambda i: (0, 0)),
            pl.BlockSpec((1, n_experts), lambda i: (0, 0)),
        ],
        out_specs=[
            pl.BlockSpec((block_t, n_experts), lambda i: (i, 0)),
            pl.BlockSpec((1, 1, 128), lambda i: (i, 0, 0)),
        ],
        out_shape=[
            jax.ShapeDtypeStruct((n_tokens, n_experts), jnp.float32),
            jax.ShapeDtypeStruct((n_tiles, 1, 128), jnp.float32),
        ],
        compiler_params=pltpu.CompilerParams(
            dimension_semantics=("arbitrary",),
            vmem_limit_bytes=110 * 1024 * 1024,
        ),
    )(x, W, b2)

    density = jnp.sum(counts[:, 0, 0]) / (n_tokens * n_experts)
    return logits, density.astype(jnp.float32)


def kernel(x, W, b):
    return _run(x, W, b, 1024)


# speedup vs baseline: 1.0088x; 1.0088x over previous
import functools

import jax
import jax.numpy as jnp
from jax.experimental import pallas as pl
from jax.experimental.pallas import tpu as pltpu

CHUNK = 256
NBUF = 4


def _router_kernel(n_chunks, x_hbm, w_ref, b_ref, out_hbm, cnt_ref, acc_ref):
    acc_ref[...] = jnp.zeros_like(acc_ref)

    def chunk_body(x_blk, out_blk):
        acc = jnp.dot(x_blk[...], w_ref[...],
                      preferred_element_type=jnp.float32)
        logits = jnp.maximum(acc + b_ref[...], 0.0)
        out_blk[...] = logits
        nz = jnp.sum((logits > 0.0).astype(jnp.float32))
        acc_ref[...] += jnp.full(acc_ref.shape, nz, dtype=jnp.float32)

    pipeline = pltpu.emit_pipeline(
        chunk_body,
        grid=(n_chunks,),
        in_specs=[
            pl.BlockSpec((CHUNK, x_hbm.shape[1]), lambda i: (i, 0),
                         pipeline_mode=pl.Buffered(buffer_count=NBUF)),
        ],
        out_specs=[
            pl.BlockSpec((CHUNK, out_hbm.shape[1]), lambda i: (i, 0)),
        ],
    )
    pipeline(x_hbm, out_hbm)
    cnt_ref[...] = acc_ref[...]


@jax.jit
def _run(x, W, b):
    n_tokens, d_model = x.shape
    n_experts = W.shape[1]
    n_chunks = n_tokens // CHUNK
    b2 = b.reshape(1, n_experts)

    logits, counts = pl.pallas_call(
        functools.partial(_router_kernel, n_chunks),
        in_specs=[
            pl.BlockSpec(memory_space=pl.ANY),
            pl.BlockSpec(memory_space=pltpu.VMEM),
            pl.BlockSpec(memory_space=pltpu.VMEM),
        ],
        out_specs=[
            pl.BlockSpec(memory_space=pl.ANY),
            pl.BlockSpec(memory_space=pltpu.VMEM),
        ],
        out_shape=[
            jax.ShapeDtypeStruct((n_tokens, n_experts), jnp.float32),
            jax.ShapeDtypeStruct((8, 128), jnp.float32),
        ],
        scratch_shapes=[
            pltpu.VMEM((8, 128), jnp.float32),
        ],
        compiler_params=pltpu.CompilerParams(
            vmem_limit_bytes=110 * 1024 * 1024,
        ),
    )(x, W, b2)

    density = counts[0, 0] / (n_tokens * n_experts)
    return logits, density.astype(jnp.float32)


def kernel(x, W, b):
    return _run(x, W, b)
